# baseline (speedup 1.0000x reference)
import jax
import jax.numpy as jnp
from jax import lax
from jax.experimental import pallas as pl
from jax.experimental.pallas import tpu as pltpu
from jax.experimental.pallas import tpu_sc as plsc

B = 4096
S = 200
D = 128
ROWS = B * S

NC = 2
NS = 16
NW = NC * NS
BAT_W = B // NW
CHUNK = S // 2
NHALF = 3
NIDX = 4


def _body(idx_hbm, tab_hbm, pos_hbm, out_hbm, idx_v, rows_v, pos_v,
          sem_i, sem_g, sem_o):
    wid = lax.axis_index("s") * NC + lax.axis_index("c")
    wbase = wid * BAT_W * S
    cbase = wid * 2 * BAT_W

    pltpu.sync_copy(pos_hbm, pos_v)

    def fire_idx(p):
        pltpu.async_copy(idx_hbm.at[pl.ds(cbase + 2 * p, 2)],
                         idx_v.at[lax.rem(p, NIDX)], sem_i)

    def wait_idx(p):
        pltpu.make_async_copy(idx_hbm.at[pl.ds(0, 2)],
                              idx_v.at[lax.rem(p, NIDX)], sem_i).wait()

    def fire_pair(p):
        half = lax.rem(p, NHALF)
        islot = lax.rem(p, NIDX)
        pltpu.async_copy(tab_hbm.at[idx_v.at[islot, 0]],
                         rows_v.at[pl.ds(half * S, CHUNK)], sem_g)
        pltpu.async_copy(tab_hbm.at[idx_v.at[islot, 1]],
                         rows_v.at[pl.ds(half * S + CHUNK, CHUNK)], sem_g)

    fire_idx(0)
    fire_idx(1)
    fire_idx(2)
    wait_idx(0)
    fire_pair(0)
    wait_idx(1)
    fire_pair(1)

    def batch_step(p, _):
        half = lax.rem(p, NHALF)
        base_v = half * S
        g = wbase + p * S

        @pl.when(p + 3 < BAT_W)
        def _():
            fire_idx(p + 3)

        pltpu.make_async_copy(tab_hbm.at[pl.ds(0, S)],
                              rows_v.at[pl.ds(base_v, S)], sem_g).wait()

        @pl.when(p >= 1)
        def _():
            pltpu.make_async_copy(
                rows_v.at[pl.ds(lax.rem(p - 1, NHALF) * S, S)],
                out_hbm.at[pl.ds(g - S, S)], sem_o).wait()

        @pl.when(p + 2 < BAT_W)
        def _():
            wait_idx(p + 2)
            fire_pair(p + 2)

        @plsc.parallel_loop(0, S, unroll=2)
        def row_step(r):
            for j in range(D // 16):
                sl = pl.ds(j * 16, 16)
                plsc.addupdate(rows_v.at[base_v + r, sl], pos_v[r, sl])

        pltpu.async_copy(rows_v.at[pl.ds(base_v, S)],
                         out_hbm.at[pl.ds(g, S)], sem_o)

        return 0

    lax.fori_loop(0, BAT_W, batch_step, 0)
    pltpu.make_async_copy(
        rows_v.at[pl.ds(lax.rem(BAT_W - 1, NHALF) * S, S)],
        out_hbm.at[pl.ds(wbase + (BAT_W - 1) * S, S)], sem_o).wait()


@jax.jit
def _embed(idx2d, response_table, position_table):
    mesh = plsc.VectorSubcoreMesh(core_axis_name="c", subcore_axis_name="s",
                                  num_cores=NC, num_subcores=NS)
    run = pl.kernel(
        _body,
        out_type=jax.ShapeDtypeStruct((ROWS, D), jnp.float32),
        mesh=mesh,
        scratch_types=[
            pltpu.VMEM((NIDX, 2, CHUNK), jnp.int32),
            pltpu.VMEM((NHALF * S, D), jnp.float32),
            pltpu.VMEM((S, D), jnp.float32),
            pltpu.SemaphoreType.DMA,
            pltpu.SemaphoreType.DMA,
            pltpu.SemaphoreType.DMA,
        ],
    )
    return run(idx2d, response_table, position_table)


def kernel(responses, response_table, position_table):
    idx2d = responses.reshape(ROWS // CHUNK, CHUNK).astype(jnp.int32)
    out = _embed(idx2d, response_table, position_table)
    return out.reshape(B, S, D)


# speedup vs baseline: 1.0660x; 1.0660x over previous
import jax
import jax.numpy as jnp
from jax import lax
from jax.experimental import pallas as pl
from jax.experimental.pallas import tpu as pltpu
from jax.experimental.pallas import tpu_sc as plsc

B = 4096
S = 200
D = 128
ROWS = B * S

NC = 2
NS = 16
NW = NC * NS
ROWS_W = ROWS // NW
CHUNK = 104
NCH = ROWS_W // CHUNK
TAIL = ROWS_W - NCH * CHUNK
TAIL_PHI = (NCH * CHUNK) % S
POS2 = S + 96
NSLOT = 6
NIDX = 8
LOOK = 2


def _body(idx_hbm, tab_hbm, pos_hbm, out_hbm, idx_v, rows_v, pos2_v,
          sem_i, sem_g, sem_o):
    wid = lax.axis_index("s") * NC + lax.axis_index("c")
    wbase = wid * ROWS_W

    pltpu.sync_copy(pos_hbm, pos2_v.at[pl.ds(0, S)])
    pltpu.sync_copy(pos_hbm.at[pl.ds(0, POS2 - S)], pos2_v.at[pl.ds(S, POS2 - S)])

    def fire_idx(c):
        pltpu.async_copy(idx_hbm.at[pl.ds(wbase + c * CHUNK, CHUNK)],
                         idx_v.at[lax.rem(c, NIDX)], sem_i)

    def wait_idx(c):
        pltpu.make_async_copy(idx_hbm.at[pl.ds(0, CHUNK)],
                              idx_v.at[lax.rem(c, NIDX)], sem_i).wait()

    def fire_gather(c):
        slot = lax.rem(c, NSLOT)
        pltpu.async_copy(tab_hbm.at[idx_v.at[lax.rem(c, NIDX)]],
                         rows_v.at[pl.ds(slot * CHUNK, CHUNK)], sem_g)

    def wait_gather(c):
        pltpu.make_async_copy(
            tab_hbm.at[pl.ds(0, CHUNK)],
            rows_v.at[pl.ds(lax.rem(c, NSLOT) * CHUNK, CHUNK)], sem_g).wait()

    def fire_out(c):
        pltpu.async_copy(rows_v.at[pl.ds(lax.rem(c, NSLOT) * CHUNK, CHUNK)],
                         out_hbm.at[pl.ds(wbase + c * CHUNK, CHUNK)], sem_o)

    def wait_out(c):
        pltpu.make_async_copy(
            rows_v.at[pl.ds(lax.rem(c, NSLOT) * CHUNK, CHUNK)],
            out_hbm.at[pl.ds(wbase + c * CHUNK, CHUNK)], sem_o).wait()

    for q in range(LOOK + 2):
        fire_idx(q)
    for q in range(LOOK):
        wait_idx(q)
        fire_gather(q)

    def chunk_step(c, _):
        slot = lax.rem(c, NSLOT)

        @pl.when(c + LOOK + 2 < NCH)
        def _():
            fire_idx(c + LOOK + 2)

        wait_gather(c)
        phi = lax.rem(c * CHUNK, S)

        @plsc.parallel_loop(0, CHUNK, unroll=2)
        def row_step(r):
            for j in range(D // 16):
                sl = pl.ds(j * 16, 16)
                plsc.addupdate(rows_v.at[slot * CHUNK + r, sl],
                               pos2_v[phi + r, sl])

        @pl.when(c >= 2)
        def _():
            wait_out(c - 2)

        fire_out(c)

        @pl.when(c + LOOK < NCH)
        def _():
            wait_idx(c + LOOK)
            fire_gather(c + LOOK)

        return 0

    lax.fori_loop(0, NCH, chunk_step, 0)
    wait_out(NCH - 2)
    wait_out(NCH - 1)

    tb = wbase + NCH * CHUNK
    pltpu.sync_copy(idx_hbm.at[pl.ds(tb, TAIL)], idx_v.at[0, pl.ds(0, TAIL)])
    pltpu.async_copy(tab_hbm.at[idx_v.at[0, pl.ds(0, TAIL)]],
                     rows_v.at[pl.ds(0, TAIL)], sem_g).wait()

    @plsc.parallel_loop(0, TAIL)
    def tail_step(r):
        for j in range(D // 16):
            sl = pl.ds(j * 16, 16)
            plsc.addupdate(rows_v.at[r, sl], pos2_v[TAIL_PHI + r, sl])

    pltpu.sync_copy(rows_v.at[pl.ds(0, TAIL)], out_hbm.at[pl.ds(tb, TAIL)])


@jax.jit
def _embed(idx_flat, response_table, position_table):
    mesh = plsc.VectorSubcoreMesh(core_axis_name="c", subcore_axis_name="s",
                                  num_cores=NC, num_subcores=NS)
    run = pl.kernel(
        _body,
        out_type=jax.ShapeDtypeStruct((ROWS, D), jnp.float32),
        mesh=mesh,
        scratch_types=[
            pltpu.VMEM((NIDX, CHUNK), jnp.int32),
            pltpu.VMEM((NSLOT * CHUNK, D), jnp.float32),
            pltpu.VMEM((POS2, D), jnp.float32),
            pltpu.SemaphoreType.DMA,
            pltpu.SemaphoreType.DMA,
            pltpu.SemaphoreType.DMA,
        ],
    )
    return run(idx_flat, response_table, position_table)


def kernel(responses, response_table, position_table):
    idx_flat = responses.reshape(ROWS).astype(jnp.int32)
    out = _embed(idx_flat, response_table, position_table)
    return out.reshape(B, S, D)


# speedup vs baseline: 1.2090x; 1.1342x over previous
import jax
import jax.numpy as jnp
from jax import lax
from jax.experimental import pallas as pl
from jax.experimental.pallas import tpu as pltpu
from jax.experimental.pallas import tpu_sc as plsc

B = 4096
S = 200
D = 128
ROWS = B * S

NC = 2
NS = 16
NW = NC * NS
BAT_W = B // NW
CHUNK = S // 2
NHALF = 3
NIDX = 4


def _body(idx_hbm, tab_hbm, pos_hbm, out_hbm, idx_v, rows_v, pos_v,
          sem_i, sem_g, sem_o):
    wid = lax.axis_index("s") * NC + lax.axis_index("c")
    wbase = wid * BAT_W * S
    cbase = wid * 2 * BAT_W

    pltpu.sync_copy(pos_hbm, pos_v)

    def fire_idx(p):
        pltpu.async_copy(idx_hbm.at[pl.ds(cbase + 2 * p, 2)],
                         idx_v.at[lax.rem(p, NIDX)], sem_i)

    def wait_idx(p):
        pltpu.make_async_copy(idx_hbm.at[pl.ds(0, 2)],
                              idx_v.at[lax.rem(p, NIDX)], sem_i).wait()

    def fire_pair(p):
        half = lax.rem(p, NHALF)
        islot = lax.rem(p, NIDX)
        pltpu.async_copy(tab_hbm.at[idx_v.at[islot, 0]],
                         rows_v.at[pl.ds(half * S, CHUNK)], sem_g)
        pltpu.async_copy(tab_hbm.at[idx_v.at[islot, 1]],
                         rows_v.at[pl.ds(half * S + CHUNK, CHUNK)], sem_g)

    fire_idx(0)
    fire_idx(1)
    fire_idx(2)
    wait_idx(0)
    fire_pair(0)
    wait_idx(1)
    fire_pair(1)

    def batch_step(p, _):
        half = lax.rem(p, NHALF)
        base_v = half * S
        g = wbase + p * S

        @pl.when(p + 3 < BAT_W)
        def _():
            fire_idx(p + 3)

        pltpu.make_async_copy(tab_hbm.at[pl.ds(0, S)],
                              rows_v.at[pl.ds(base_v, S)], sem_g).wait()

        @plsc.parallel_loop(0, S, unroll=2)
        def row_step(r):
            for j in range(D // 16):
                sl = pl.ds(j * 16, 16)
                plsc.addupdate(rows_v.at[base_v + r, sl], pos_v[r, sl])

        @pl.when(p >= 1)
        def _():
            pltpu.make_async_copy(
                rows_v.at[pl.ds(lax.rem(p - 1, NHALF) * S, S)],
                out_hbm.at[pl.ds(g - S, S)], sem_o).wait()

        pltpu.async_copy(rows_v.at[pl.ds(base_v, S)],
                         out_hbm.at[pl.ds(g, S)], sem_o)

        @pl.when(p + 2 < BAT_W)
        def _():
            wait_idx(p + 2)
            fire_pair(p + 2)

        return 0

    lax.fori_loop(0, BAT_W, batch_step, 0)
    pltpu.make_async_copy(
        rows_v.at[pl.ds(lax.rem(BAT_W - 1, NHALF) * S, S)],
        out_hbm.at[pl.ds(wbase + (BAT_W - 1) * S, S)], sem_o).wait()


@jax.jit
def _embed(idx2d, response_table, position_table):
    mesh = plsc.VectorSubcoreMesh(core_axis_name="c", subcore_axis_name="s",
                                  num_cores=NC, num_subcores=NS)
    run = pl.kernel(
        _body,
        out_type=jax.ShapeDtypeStruct((ROWS, D), jnp.float32),
        mesh=mesh,
        scratch_types=[
            pltpu.VMEM((NIDX, 2, CHUNK), jnp.int32),
            pltpu.VMEM((NHALF * S, D), jnp.float32),
            pltpu.VMEM((S, D), jnp.float32),
            pltpu.SemaphoreType.DMA,
            pltpu.SemaphoreType.DMA,
            pltpu.SemaphoreType.DMA,
        ],
    )
    return run(idx2d, response_table, position_table)


def kernel(responses, response_table, position_table):
    idx2d = responses.reshape(ROWS // CHUNK, CHUNK).astype(jnp.int32)
    out = _embed(idx2d, response_table, position_table)
    return out.reshape(B, S, D)
